# Initial kernel scaffold; baseline (speedup 1.0000x reference)
import jax
import jax.numpy as jnp
from jax.experimental import pallas as pl


def kernel(x, table):
    raise NotImplementedError("write your pallas kernel here")


# speedup vs baseline: 1.8442x; 1.8442x over previous
import functools

import jax
import jax.numpy as jnp
from jax import lax
from jax.experimental import pallas as pl
from jax.experimental.pallas import tpu as pltpu
from jax.experimental.pallas import tpu_sc as plsc

_NC = 2
_NS = 16
_NW = _NC * _NS

_IDXW = 128
_KROWS = 8


@functools.lru_cache(maxsize=None)
def _make_gather(n_rows, d_model):
    assert n_rows % _NW == 0
    rows_per_w = n_rows // _NW
    assert rows_per_w % _KROWS == 0
    n_chunks = rows_per_w // _KROWS

    mesh = plsc.VectorSubcoreMesh(core_axis_name="c", subcore_axis_name="s")

    @functools.partial(
        pl.kernel,
        mesh=mesh,
        out_type=jax.ShapeDtypeStruct((n_rows, _IDXW, d_model), jnp.float32),
        compiler_params=pltpu.CompilerParams(use_tc_tiling_on_sc=False),
        scratch_types=[
            pltpu.VMEM((_KROWS, _IDXW), jnp.int32),
            pltpu.VMEM((_KROWS, _IDXW, d_model), jnp.float32),
            pltpu.SemaphoreType.DMA,
        ],
    )
    def gather_kernel(x_hbm, table_hbm, out_hbm, idx_v, rows_v, sem):
        wid = lax.axis_index("s") * _NC + lax.axis_index("c")
        row_base = wid * rows_per_w

        def chunk(i, carry):
            r0 = row_base + i * _KROWS
            pltpu.sync_copy(x_hbm.at[pl.ds(r0, _KROWS)], idx_v)
            copies = []
            for j in range(_KROWS):
                copies.append(
                    pltpu.async_copy(table_hbm.at[idx_v.at[j]], rows_v.at[j], sem)
                )
            for c in copies:
                c.wait()
            pltpu.sync_copy(rows_v, out_hbm.at[pl.ds(r0, _KROWS)])
            return carry

        lax.fori_loop(0, n_chunks, chunk, 0)

    return gather_kernel


def kernel(x, table):
    batch, hist = x.shape
    d_model = table.shape[1]
    n_idx = batch * hist
    n_rows = n_idx // _IDXW
    x2d = x.reshape(n_rows, _IDXW).astype(jnp.int32)
    out = _make_gather(n_rows, d_model)(x2d, table)
    return out.reshape(batch, hist, d_model)


# speedup vs baseline: 1.8727x; 1.0155x over previous
import functools

import jax
import jax.numpy as jnp
from jax import lax
from jax.experimental import pallas as pl
from jax.experimental.pallas import tpu as pltpu
from jax.experimental.pallas import tpu_sc as plsc

_NC = 2
_NS = 16
_NW = _NC * _NS

_IDXW = 128
_KROWS = 4
_NBUF = 2


@functools.lru_cache(maxsize=None)
def _make_gather(n_rows, d_model):
    assert n_rows % _NW == 0
    rows_per_w = n_rows // _NW
    assert rows_per_w % (_KROWS * _NBUF) == 0
    n_super = rows_per_w // (_KROWS * _NBUF)

    mesh = plsc.VectorSubcoreMesh(core_axis_name="c", subcore_axis_name="s")

    @functools.partial(
        pl.kernel,
        mesh=mesh,
        out_type=jax.ShapeDtypeStruct((n_rows, _IDXW, d_model), jnp.float32),
        compiler_params=pltpu.CompilerParams(use_tc_tiling_on_sc=False),
        scratch_types=[
            pltpu.VMEM((_NBUF, _KROWS, _IDXW), jnp.int32),
            pltpu.VMEM((_NBUF, _KROWS, _IDXW, d_model), jnp.float32),
            pltpu.SemaphoreType.DMA,
            pltpu.SemaphoreType.DMA,
            pltpu.SemaphoreType.DMA,
            pltpu.SemaphoreType.DMA,
        ],
    )
    def gather_kernel(x_hbm, table_hbm, out_hbm, idx_v, rows_v, g0, g1, o0, o1):
        gsem = [g0, g1]
        osem = [o0, o1]
        wid = lax.axis_index("s") * _NC + lax.axis_index("c")
        row_base = wid * rows_per_w

        def super_iter(t, carry):
            for b in range(_NBUF):
                r0 = row_base + (t * _NBUF + b) * _KROWS

                @pl.when(t > 0)
                def _():
                    pltpu.make_async_copy(
                        rows_v.at[b], out_hbm.at[pl.ds(r0, _KROWS)], osem[b]
                    ).wait()

                pltpu.sync_copy(x_hbm.at[pl.ds(r0, _KROWS)], idx_v.at[b])
                for j in range(_KROWS):
                    pltpu.async_copy(
                        table_hbm.at[idx_v.at[b].at[j]], rows_v.at[b].at[j], gsem[b]
                    )

            for b in range(_NBUF):
                r0 = row_base + (t * _NBUF + b) * _KROWS
                for j in range(_KROWS):
                    pltpu.make_async_copy(
                        table_hbm.at[idx_v.at[b].at[j]], rows_v.at[b].at[j], gsem[b]
                    ).wait()
                pltpu.async_copy(rows_v.at[b], out_hbm.at[pl.ds(r0, _KROWS)], osem[b])
            return carry

        lax.fori_loop(0, n_super, super_iter, 0)

        for b in range(_NBUF):
            r0 = row_base + ((n_super - 1) * _NBUF + b) * _KROWS
            pltpu.make_async_copy(
                rows_v.at[b], out_hbm.at[pl.ds(r0, _KROWS)], osem[b]
            ).wait()

    return gather_kernel


def kernel(x, table):
    batch, hist = x.shape
    d_model = table.shape[1]
    n_idx = batch * hist
    n_rows = n_idx // _IDXW
    x2d = x.reshape(n_rows, _IDXW).astype(jnp.int32)
    out = _make_gather(n_rows, d_model)(x2d, table)
    return out.reshape(batch, hist, d_model)
